# Initial kernel scaffold; baseline (speedup 1.0000x reference)
import jax
import jax.numpy as jnp
from jax import lax
from jax.experimental import pallas as pl
from jax.experimental.pallas import tpu as pltpu
from jax.experimental.pallas import tpu_sc as plsc

_ANGSTROM_TO_BOHR = 1.8897261258369282
_CUTOFF = 5.2

_NC, _NS, _L = 2, 16, 16
_NW = _NC * _NS
_G = 80
_NG = 25
_CH = _G * _NG


def _sqrt_newton(x):
    i = plsc.bitcast(x, jnp.int32)
    y = plsc.bitcast(jnp.int32(0x5F3759DF) - (i >> 1), jnp.float32)
    for _ in range(3):
        y = y * (1.5 - 0.5 * x * y * y)
    return x * y


def _make_sc_kernel(n_mol, n_atoms, n_pairs):
    assert n_pairs % (_NW * _CH) == 0
    nchunk = n_pairs // (_NW * _CH)
    nb = ((n_mol + _L - 1) // _L) * _L

    mesh = plsc.VectorSubcoreMesh(
        core_axis_name="c", subcore_axis_name="s",
        num_cores=_NC, num_subcores=_NS)

    rc = _CUTOFF * _ANGSTROM_TO_BOHR
    inv_rc = 1.0 / rc
    inv_atoms = 1.0 / n_atoms

    def body(elem_hbm, i0_hbm, i1_hbm, d_hbm, y_hbm, sa_hbm, kr_hbm, out_hbm,
             i0_v, i1_v, s0_v, s1_v, d_v, bins, part, tmp,
             y16, sa16, kr16, elem_sh, stage_sh, sem, gsem):
        cid = lax.axis_index("c")
        sid = lax.axis_index("s")
        w = cid * _NS + sid

        pltpu.sync_copy(y_hbm, y16)
        pltpu.sync_copy(sa_hbm, sa16)
        pltpu.sync_copy(kr_hbm, kr16)

        @pl.when(sid == 0)
        def _():
            pltpu.sync_copy(elem_hbm, elem_sh)

        zeros16 = jnp.zeros((_L,), jnp.float32)

        @pl.loop(0, nb, step=_L)
        def _(k):
            for l in range(_L):
                bins[l, pl.ds(k, _L)] = zeros16

        plsc.subcore_barrier()

        lane = lax.iota(jnp.int32, _L)

        @pl.loop(0, nchunk)
        def _(ch):
            gchunk = w * nchunk + ch
            cp0 = pltpu.async_copy(i0_hbm.at[gchunk], i0_v, sem)
            cp1 = pltpu.async_copy(i1_hbm.at[gchunk], i1_v, sem)
            cp2 = pltpu.async_copy(d_hbm.at[gchunk], d_v, sem)
            cp0.wait()
            cp1.wait()
            cp2.wait()
            gathers = []
            for g in range(_NG):
                gathers.append(pltpu.async_copy(
                    elem_sh.at[i0_v.at[g]], s0_v.at[g], gsem))
                gathers.append(pltpu.async_copy(
                    elem_sh.at[i1_v.at[g]], s1_v.at[g], gsem))
            for cp in gathers:
                cp.wait()

            @pl.loop(0, _NG)
            def _(g):
                for c in range(0, _G, _L):
                    i0 = i0_v[g, pl.ds(c, _L)]
                    s0 = s0_v[g, pl.ds(c, _L)]
                    s1 = s1_v[g, pl.ds(c, _L)]
                    dist = d_v[pl.ds(g * _G + c, _L)]
                    d = jnp.maximum(dist, 1e-7) * _ANGSTROM_TO_BOHR
                    p = s0 * 4 + s1
                    y = plsc.load_gather(y16, [p])
                    sa = plsc.load_gather(sa16, [p])
                    kr = plsc.load_gather(kr16, [p])
                    term = jnp.where(kr > 1.25, d * _sqrt_newton(d), d)
                    m = d * inv_rc
                    msafe = jnp.minimum(m, 0.9999999)
                    fcarg = 1.0 - 1.0 / (1.0 - msafe * msafe)
                    e = (y / d) * jnp.exp(fcarg - sa * term)
                    e = jnp.where(m < 1.0, e, 0.0)
                    mol = ((i0.astype(jnp.float32) + 0.5)
                           * inv_atoms).astype(jnp.int32)
                    plsc.addupdate_scatter(bins, [lane, mol], e)

        @pl.loop(0, nb, step=_L)
        def _(k):
            acc = bins[0, pl.ds(k, _L)]
            for l in range(1, _L):
                acc = acc + bins[l, pl.ds(k, _L)]
            part[pl.ds(k, _L)] = acc

        pltpu.sync_copy(part, stage_sh.at[sid])
        plsc.subcore_barrier()

        @pl.when(sid == 0)
        def _():
            pltpu.sync_copy(stage_sh.at[0], part)
            for s in range(1, _NS):
                pltpu.sync_copy(stage_sh.at[s], tmp)

                @pl.loop(0, nb, step=_L)
                def _(k):
                    part[pl.ds(k, _L)] = part[pl.ds(k, _L)] + tmp[pl.ds(k, _L)]
            pltpu.sync_copy(part, out_hbm.at[cid])

    return pl.kernel(
        body,
        out_type=jax.ShapeDtypeStruct((_NC, nb), jnp.float32),
        mesh=mesh,
        scratch_types=[
            pltpu.VMEM((_NG, _G), jnp.int32),
            pltpu.VMEM((_NG, _G), jnp.int32),
            pltpu.VMEM((_NG, _G), jnp.int32),
            pltpu.VMEM((_NG, _G), jnp.int32),
            pltpu.VMEM((_CH,), jnp.float32),
            pltpu.VMEM((_L, nb), jnp.float32),
            pltpu.VMEM((nb,), jnp.float32),
            pltpu.VMEM((nb,), jnp.float32),
            pltpu.VMEM((16,), jnp.float32),
            pltpu.VMEM((16,), jnp.float32),
            pltpu.VMEM((16,), jnp.float32),
            pltpu.VMEM_SHARED((n_mol * n_atoms,), jnp.int32),
            pltpu.VMEM_SHARED((_NS, nb), jnp.float32),
            pltpu.SemaphoreType.DMA,
            pltpu.SemaphoreType.DMA,
        ],
    )


@jax.jit
def kernel(element_idxs, neighbor_idxs, distances, y_ab, sqrt_alpha_ab,
           k_rep_ab):
    n_mol, n_atoms = element_idxs.shape
    n_pairs = distances.shape[0]
    elem_flat = element_idxs.reshape(-1)
    i0 = neighbor_idxs[0].reshape(-1, _NG, _G)
    i1 = neighbor_idxs[1].reshape(-1, _NG, _G)
    d = distances.reshape(-1, _CH)
    sc = _make_sc_kernel(n_mol, n_atoms, n_pairs)
    out = sc(elem_flat, i0, i1, d, y_ab.reshape(-1), sqrt_alpha_ab.reshape(-1),
             k_rep_ab.reshape(-1))
    return out[0, :n_mol] + out[1, :n_mol]


# speedup vs baseline: 323.9029x; 323.9029x over previous
import dataclasses

import jax
import jax.numpy as jnp
from jax import lax
from jax.experimental import pallas as pl
from jax.experimental.pallas import tpu as pltpu
from jax.experimental.pallas import tpu_sc as plsc

_ANGSTROM_TO_BOHR = 1.8897261258369282
_CUTOFF = 5.2

_NC, _NS, _L = 2, 16, 16
_NW = _NC * _NS
_G = 80
_NG = 25
_CH = _G * _NG


def _rcp(x):
    r = 1.0 / x
    r = r * (2.0 - x * r)
    r = r * (2.0 - x * r)
    return r


def _sqrt_newton(x):
    i = plsc.bitcast(x, jnp.int32)
    y = plsc.bitcast(jnp.int32(0x5F3759DF) - (i >> 1), jnp.float32)
    for _ in range(3):
        y = y * (1.5 - 0.5 * x * y * y)
    return x * y


def _make_sc_kernel(n_mol, n_atoms, n_pairs):
    assert n_pairs % (_NW * _CH) == 0
    nchunk = n_pairs // (_NW * _CH)
    nb = ((n_mol + _L - 1) // _L) * _L

    mesh = plsc.VectorSubcoreMesh(
        core_axis_name="c", subcore_axis_name="s",
        num_cores=_NC, num_subcores=_NS)

    rc = _CUTOFF * _ANGSTROM_TO_BOHR
    inv_rc = 1.0 / rc
    inv_atoms = 1.0 / n_atoms

    def body(elem_hbm, i0_hbm, i1_hbm, d_hbm, y_hbm, sa_hbm, kr_hbm, out_hbm,
             i0_v, i1_v, s0_v, s1_v, d_v, bins, part, tmp,
             y16, sa16, kr16, elem_sh, stage_sh, sem, gsem):
        cid = lax.axis_index("c")
        sid = lax.axis_index("s")
        w = cid * _NS + sid

        pltpu.sync_copy(y_hbm, y16)
        pltpu.sync_copy(sa_hbm, sa16)
        pltpu.sync_copy(kr_hbm, kr16)

        @pl.when(sid == 0)
        def _():
            pltpu.sync_copy(elem_hbm, elem_sh)

        zeros16 = jnp.zeros((_L,), jnp.float32)

        @pl.loop(0, nb, step=_L)
        def _(k):
            for l in range(_L):
                bins[l, pl.ds(k, _L)] = zeros16

        plsc.subcore_barrier()

        lane = lax.iota(jnp.int32, _L)

        @pl.loop(0, nchunk)
        def _(ch):
            gchunk = w * nchunk + ch
            cp0 = pltpu.async_copy(i0_hbm.at[gchunk], i0_v, sem)
            cp1 = pltpu.async_copy(i1_hbm.at[gchunk], i1_v, sem)
            cp2 = pltpu.async_copy(d_hbm.at[gchunk], d_v, sem)
            cp0.wait()
            cp1.wait()
            cp2.wait()
            gathers = []
            for g in range(_NG):
                gathers.append(pltpu.async_copy(
                    elem_sh.at[i0_v.at[g]], s0_v.at[g], gsem))
                gathers.append(pltpu.async_copy(
                    elem_sh.at[i1_v.at[g]], s1_v.at[g], gsem))
            for cp in gathers:
                cp.wait()

            @pl.loop(0, _NG)
            def _(g):
                for c in range(0, _G, _L):
                    i0 = i0_v[g, pl.ds(c, _L)]
                    s0 = s0_v[g, pl.ds(c, _L)]
                    s1 = s1_v[g, pl.ds(c, _L)]
                    dist = d_v[pl.ds(g * _G + c, _L)]
                    d = jnp.maximum(dist, 1e-7) * _ANGSTROM_TO_BOHR
                    p = s0 * 4 + s1
                    y = plsc.load_gather(y16, [p])
                    sa = plsc.load_gather(sa16, [p])
                    kr = plsc.load_gather(kr16, [p])
                    term = jnp.where(kr > 1.25, d * _sqrt_newton(d), d)
                    m = d * inv_rc
                    msafe = jnp.minimum(m, 0.9999999)
                    fcarg = 1.0 - _rcp(1.0 - msafe * msafe)
                    e = y * _rcp(d) * jnp.exp(fcarg - sa * term)
                    e = jnp.where(m < 1.0, e, 0.0)
                    mol = ((i0.astype(jnp.float32) + 0.5)
                           * inv_atoms).astype(jnp.int32)
                    plsc.addupdate_scatter(bins, [lane, mol], e)

        @pl.loop(0, nb, step=_L)
        def _(k):
            acc = bins[0, pl.ds(k, _L)]
            for l in range(1, _L):
                acc = acc + bins[l, pl.ds(k, _L)]
            part[pl.ds(k, _L)] = acc

        plsc.subcore_barrier()
        pltpu.sync_copy(part, stage_sh.at[sid])
        plsc.subcore_barrier()

        @pl.when(sid == 0)
        def _():
            pltpu.sync_copy(stage_sh.at[0], part)
            for s in range(1, _NS):
                pltpu.sync_copy(stage_sh.at[s], tmp)

                @pl.loop(0, nb, step=_L)
                def _(k):
                    part[pl.ds(k, _L)] = part[pl.ds(k, _L)] + tmp[pl.ds(k, _L)]
            pltpu.sync_copy(part, out_hbm.at[cid])

    cp = pltpu.CompilerParams()
    if "needs_layout_passes" in pltpu.CompilerParams.__dataclass_fields__:
        cp = dataclasses.replace(cp, needs_layout_passes=False)

    return pl.kernel(
        body,
        out_type=jax.ShapeDtypeStruct((_NC, nb), jnp.float32),
        mesh=mesh,
        compiler_params=cp,
        scratch_types=[
            pltpu.VMEM((_NG, _G), jnp.int32),
            pltpu.VMEM((_NG, _G), jnp.int32),
            pltpu.VMEM((_NG, _G), jnp.int32),
            pltpu.VMEM((_NG, _G), jnp.int32),
            pltpu.VMEM((_CH,), jnp.float32),
            pltpu.VMEM((_L, nb), jnp.float32),
            pltpu.VMEM((nb,), jnp.float32),
            pltpu.VMEM((nb,), jnp.float32),
            pltpu.VMEM((16,), jnp.float32),
            pltpu.VMEM((16,), jnp.float32),
            pltpu.VMEM((16,), jnp.float32),
            pltpu.VMEM_SHARED((n_mol * n_atoms,), jnp.int32),
            pltpu.VMEM_SHARED((_NS, nb), jnp.float32),
            pltpu.SemaphoreType.DMA,
            pltpu.SemaphoreType.DMA,
        ],
    )


@jax.jit
def kernel(element_idxs, neighbor_idxs, distances, y_ab, sqrt_alpha_ab,
           k_rep_ab):
    n_mol, n_atoms = element_idxs.shape
    n_pairs = distances.shape[0]
    elem_flat = element_idxs.reshape(-1)
    i0 = neighbor_idxs[0].reshape(-1, _NG, _G)
    i1 = neighbor_idxs[1].reshape(-1, _NG, _G)
    d = distances.reshape(-1, _CH)
    sc = _make_sc_kernel(n_mol, n_atoms, n_pairs)
    out = sc(elem_flat, i0, i1, d, y_ab.reshape(-1), sqrt_alpha_ab.reshape(-1),
             k_rep_ab.reshape(-1))
    return out[0, :n_mol] + out[1, :n_mol]


# speedup vs baseline: 412.7324x; 1.2742x over previous
import dataclasses

import jax
import jax.numpy as jnp
from jax import lax
from jax.experimental import pallas as pl
from jax.experimental.pallas import tpu as pltpu
from jax.experimental.pallas import tpu_sc as plsc

_ANGSTROM_TO_BOHR = 1.8897261258369282
_CUTOFF = 5.2

_NC, _NS, _L = 2, 16, 16
_NW = _NC * _NS
_CH = 2000


def _rcp(x):
    r = 1.0 / x
    r = r * (2.0 - x * r)
    r = r * (2.0 - x * r)
    return r


def _sqrt_newton(x):
    i = plsc.bitcast(x, jnp.int32)
    y = plsc.bitcast(jnp.int32(0x5F3759DF) - (i >> 1), jnp.float32)
    for _ in range(3):
        y = y * (1.5 - 0.5 * x * y * y)
    return x * y


def _make_sc_kernel(n_mol, n_atoms, n_pairs):
    assert n_pairs % (_NW * _CH * 2) == 0
    nchunk = n_pairs // (_NW * _CH)
    nb = ((n_mol + _L - 1) // _L) * _L

    mesh = plsc.VectorSubcoreMesh(
        core_axis_name="c", subcore_axis_name="s",
        num_cores=_NC, num_subcores=_NS)

    inv_rc = 1.0 / (_CUTOFF * _ANGSTROM_TO_BOHR)
    inv_atoms = 1.0 / n_atoms

    def body(elem_hbm, i0_hbm, i1_hbm, d_hbm, y_hbm, sa_hbm, kr_hbm, out_hbm,
             elem_v, i0a, i1a, da, i0b, i1b, db, bins, part,
             y16, sa16, kr16, sema, semb):
        cid = lax.axis_index("c")
        sid = lax.axis_index("s")
        w = cid * _NS + sid
        base = w * nchunk

        pltpu.sync_copy(y_hbm, y16)
        pltpu.sync_copy(sa_hbm, sa16)
        pltpu.sync_copy(kr_hbm, kr16)
        pltpu.sync_copy(elem_hbm, elem_v)

        zeros16 = jnp.zeros((_L,), jnp.float32)

        @pl.loop(0, nb, step=_L)
        def _(k):
            for l in range(_L):
                bins[l, pl.ds(k, _L)] = zeros16

        lane = lax.iota(jnp.int32, _L)

        def lin(bufs, ch, sem):
            return [pltpu.make_async_copy(i0_hbm.at[ch], bufs[0], sem),
                    pltpu.make_async_copy(i1_hbm.at[ch], bufs[1], sem),
                    pltpu.make_async_copy(d_hbm.at[ch], bufs[2], sem)]

        def fire(bufs, ch, sem):
            for c in lin(bufs, ch, sem):
                c.start()

        def drain(bufs, ch, sem):
            for c in lin(bufs, ch, sem):
                c.wait()

        def compute(bufs):
            i0_v, i1_v, d_v = bufs

            @pl.loop(0, _CH, step=_L)
            def _(ci):
                for half in range(1):
                    sl = pl.ds(ci + half * _L, _L)
                    i0 = i0_v[sl]
                    s0 = plsc.load_gather(elem_v, [i0])
                    s1 = plsc.load_gather(elem_v, [i1_v[sl]])
                    d = jnp.maximum(d_v[sl], 1e-7) * _ANGSTROM_TO_BOHR
                    p = s0 * 4 + s1
                    y = plsc.load_gather(y16, [p])
                    sa = plsc.load_gather(sa16, [p])
                    kr = plsc.load_gather(kr16, [p])
                    term = jnp.where(kr > 1.25, d * _sqrt_newton(d), d)
                    m = d * inv_rc
                    msafe = jnp.minimum(m, 0.9999999)
                    fcarg = 1.0 - _rcp(1.0 - msafe * msafe)
                    e = y * _rcp(d) * jnp.exp(fcarg - sa * term)
                    e = jnp.where(m < 1.0, e, 0.0)
                    mol = ((i0.astype(jnp.float32) + 0.5)
                           * inv_atoms).astype(jnp.int32)
                    plsc.addupdate_scatter(bins, [lane, mol], e)

        bufa = (i0a, i1a, da)
        bufb = (i0b, i1b, db)
        fire(bufa, base, sema)

        @pl.loop(0, nchunk // 2)
        def _(t):
            ca = base + 2 * t
            drain(bufa, ca, sema)
            fire(bufb, ca + 1, semb)
            compute(bufa)
            drain(bufb, ca + 1, semb)

            @pl.when(t < nchunk // 2 - 1)
            def _():
                fire(bufa, ca + 2, sema)
            compute(bufb)

        @pl.loop(0, nb, step=_L)
        def _(k):
            acc = bins[0, pl.ds(k, _L)]
            for l in range(1, _L):
                acc = acc + bins[l, pl.ds(k, _L)]
            part[pl.ds(k, _L)] = acc

        pltpu.sync_copy(part, out_hbm.at[w])

    cp = pltpu.CompilerParams()
    if "needs_layout_passes" in pltpu.CompilerParams.__dataclass_fields__:
        cp = dataclasses.replace(cp, needs_layout_passes=False)

    return pl.kernel(
        body,
        out_type=jax.ShapeDtypeStruct((_NW, nb), jnp.float32),
        mesh=mesh,
        compiler_params=cp,
        scratch_types=[
            pltpu.VMEM((n_mol * n_atoms,), jnp.int32),
            pltpu.VMEM((_CH,), jnp.int32),
            pltpu.VMEM((_CH,), jnp.int32),
            pltpu.VMEM((_CH,), jnp.float32),
            pltpu.VMEM((_CH,), jnp.int32),
            pltpu.VMEM((_CH,), jnp.int32),
            pltpu.VMEM((_CH,), jnp.float32),
            pltpu.VMEM((_L, nb), jnp.float32),
            pltpu.VMEM((nb,), jnp.float32),
            pltpu.VMEM((16,), jnp.float32),
            pltpu.VMEM((16,), jnp.float32),
            pltpu.VMEM((16,), jnp.float32),
            pltpu.SemaphoreType.DMA,
            pltpu.SemaphoreType.DMA,
        ],
    )


@jax.jit
def kernel(element_idxs, neighbor_idxs, distances, y_ab, sqrt_alpha_ab,
           k_rep_ab):
    n_mol, n_atoms = element_idxs.shape
    n_pairs = distances.shape[0]
    elem_flat = element_idxs.reshape(-1)
    i0 = neighbor_idxs[0].reshape(-1, _CH)
    i1 = neighbor_idxs[1].reshape(-1, _CH)
    d = distances.reshape(-1, _CH)
    sc = _make_sc_kernel(n_mol, n_atoms, n_pairs)
    out = sc(elem_flat, i0, i1, d, y_ab.reshape(-1), sqrt_alpha_ab.reshape(-1),
             k_rep_ab.reshape(-1))
    return jnp.sum(out, axis=0)[:n_mol]


# speedup vs baseline: 440.6587x; 1.0677x over previous
import dataclasses

import jax
import jax.numpy as jnp
from jax import lax
from jax.experimental import pallas as pl
from jax.experimental.pallas import tpu as pltpu
from jax.experimental.pallas import tpu_sc as plsc

_ANGSTROM_TO_BOHR = 1.8897261258369282
_CUTOFF = 5.2

_NC, _NS, _L = 2, 16, 16
_NW = _NC * _NS
_CH = 2000


def _rcp(x):
    r = 1.0 / x
    r = r * (2.0 - x * r)
    return r


def _sqrt_newton(x):
    i = plsc.bitcast(x, jnp.int32)
    y = plsc.bitcast(jnp.int32(0x5F3759DF) - (i >> 1), jnp.float32)
    for _ in range(2):
        y = y * (1.5 - 0.5 * x * y * y)
    return x * y


def _make_sc_kernel(n_mol, n_atoms, n_pairs):
    assert n_pairs % (_NW * _CH * 2) == 0
    nchunk = n_pairs // (_NW * _CH)
    nb = ((n_mol + _L - 1) // _L) * _L

    mesh = plsc.VectorSubcoreMesh(
        core_axis_name="c", subcore_axis_name="s",
        num_cores=_NC, num_subcores=_NS)

    inv_rc = 1.0 / (_CUTOFF * _ANGSTROM_TO_BOHR)
    inv_atoms = 1.0 / n_atoms

    def body(elem_hbm, i0_hbm, i1_hbm, d_hbm, y_hbm, sa_hbm, kr_hbm, out_hbm,
             elem_v, i0a, i1a, da, i0b, i1b, db, bins, part,
             y16, sa16, kr16, sema, semb):
        cid = lax.axis_index("c")
        sid = lax.axis_index("s")
        w = cid * _NS + sid
        base = w * nchunk

        pltpu.sync_copy(y_hbm, y16)
        pltpu.sync_copy(sa_hbm, sa16)
        pltpu.sync_copy(kr_hbm, kr16)
        pltpu.sync_copy(elem_hbm, elem_v)

        zeros16 = jnp.zeros((_L,), jnp.float32)

        @pl.loop(0, nb, step=_L)
        def _(k):
            for l in range(_L):
                bins[l, pl.ds(k, _L)] = zeros16

        lane = lax.iota(jnp.int32, _L)

        def lin(bufs, ch, sem):
            return [pltpu.make_async_copy(i0_hbm.at[ch], bufs[0], sem),
                    pltpu.make_async_copy(i1_hbm.at[ch], bufs[1], sem),
                    pltpu.make_async_copy(d_hbm.at[ch], bufs[2], sem)]

        def fire(bufs, ch, sem):
            for c in lin(bufs, ch, sem):
                c.start()

        def drain(bufs, ch, sem):
            for c in lin(bufs, ch, sem):
                c.wait()

        def compute(bufs):
            i0_v, i1_v, d_v = bufs

            def block(sl):
                i0 = i0_v[sl]
                s0 = plsc.load_gather(elem_v, [i0])
                s1 = plsc.load_gather(elem_v, [i1_v[sl]])
                d = jnp.maximum(d_v[sl], 1e-7) * _ANGSTROM_TO_BOHR
                p = s0 * 4 + s1
                y = plsc.load_gather(y16, [p])
                sa = plsc.load_gather(sa16, [p])
                kr = plsc.load_gather(kr16, [p])
                term = jnp.where(kr > 1.25, d * _sqrt_newton(d), d)
                m = d * inv_rc
                msafe = jnp.minimum(m, 0.9999999)
                fcarg = 1.0 - _rcp(1.0 - msafe * msafe)
                e = y * _rcp(d) * jnp.exp(fcarg - sa * term)
                e = jnp.where(m < 1.0, e, 0.0)
                mol = ((i0.astype(jnp.float32) + 0.5)
                       * inv_atoms).astype(jnp.int32)
                plsc.addupdate_scatter(bins, [lane, mol], e)

            unroll = 4
            main = (_CH // (unroll * _L)) * (unroll * _L)

            @pl.loop(0, main, step=unroll * _L)
            def _(ci):
                for u in range(unroll):
                    block(pl.ds(ci + u * _L, _L))

            for ci in range(main, _CH, _L):
                block(pl.ds(ci, _L))

        bufa = (i0a, i1a, da)
        bufb = (i0b, i1b, db)
        fire(bufa, base, sema)

        @pl.loop(0, nchunk // 2)
        def _(t):
            ca = base + 2 * t
            drain(bufa, ca, sema)
            fire(bufb, ca + 1, semb)
            compute(bufa)
            drain(bufb, ca + 1, semb)

            @pl.when(t < nchunk // 2 - 1)
            def _():
                fire(bufa, ca + 2, sema)
            compute(bufb)

        @pl.loop(0, nb, step=_L)
        def _(k):
            acc = bins[0, pl.ds(k, _L)]
            for l in range(1, _L):
                acc = acc + bins[l, pl.ds(k, _L)]
            part[pl.ds(k, _L)] = acc

        pltpu.sync_copy(part, out_hbm.at[w])

    cp = pltpu.CompilerParams()
    if "needs_layout_passes" in pltpu.CompilerParams.__dataclass_fields__:
        cp = dataclasses.replace(cp, needs_layout_passes=False)

    return pl.kernel(
        body,
        out_type=jax.ShapeDtypeStruct((_NW, nb), jnp.float32),
        mesh=mesh,
        compiler_params=cp,
        scratch_types=[
            pltpu.VMEM((n_mol * n_atoms,), jnp.int32),
            pltpu.VMEM((_CH,), jnp.int32),
            pltpu.VMEM((_CH,), jnp.int32),
            pltpu.VMEM((_CH,), jnp.float32),
            pltpu.VMEM((_CH,), jnp.int32),
            pltpu.VMEM((_CH,), jnp.int32),
            pltpu.VMEM((_CH,), jnp.float32),
            pltpu.VMEM((_L, nb), jnp.float32),
            pltpu.VMEM((nb,), jnp.float32),
            pltpu.VMEM((16,), jnp.float32),
            pltpu.VMEM((16,), jnp.float32),
            pltpu.VMEM((16,), jnp.float32),
            pltpu.SemaphoreType.DMA,
            pltpu.SemaphoreType.DMA,
        ],
    )


@jax.jit
def kernel(element_idxs, neighbor_idxs, distances, y_ab, sqrt_alpha_ab,
           k_rep_ab):
    n_mol, n_atoms = element_idxs.shape
    n_pairs = distances.shape[0]
    elem_flat = element_idxs.reshape(-1)
    i0 = neighbor_idxs[0].reshape(-1, _CH)
    i1 = neighbor_idxs[1].reshape(-1, _CH)
    d = distances.reshape(-1, _CH)
    sc = _make_sc_kernel(n_mol, n_atoms, n_pairs)
    out = sc(elem_flat, i0, i1, d, y_ab.reshape(-1), sqrt_alpha_ab.reshape(-1),
             k_rep_ab.reshape(-1))
    return jnp.sum(out, axis=0)[:n_mol]
